# Initial kernel scaffold; baseline (speedup 1.0000x reference)
import jax
import jax.numpy as jnp
from jax.experimental import pallas as pl


def kernel(x, edge_index, edge_attr, params):
    raise NotImplementedError("write your pallas kernel here")


# speedup vs baseline: 1.7144x; 1.7144x over previous
import functools

import jax
import jax.numpy as jnp
from jax import lax
from jax.experimental import pallas as pl
from jax.experimental.pallas import tpu as pltpu
from jax.experimental.pallas import tpu_sc as plsc

N = 10000
E = 160000
H = 128
ALPHA = 0.5

_NC = 2
_NS = 16
_NW = _NC * _NS


def _mm(a, b):
    return jax.lax.dot_general(a, b, (((1,), (0,)), ((), ())),
                               preferred_element_type=jnp.float32)


@functools.partial(jax.jit, static_argnames=("ni",))
def _sc_gather(table, idx3d, ni):
    per_w = ni // _NW
    chunk = 1000
    n_chunks = per_w // chunk
    mesh = plsc.VectorSubcoreMesh(core_axis_name="c", subcore_axis_name="s")

    @functools.partial(
        pl.kernel, mesh=mesh,
        compiler_params=pltpu.CompilerParams(use_tc_tiling_on_sc=False),
        out_type=jax.ShapeDtypeStruct((ni, H), jnp.float32),
        scratch_types=[
            pltpu.VMEM((8, 125), jnp.int32),
            pltpu.VMEM((chunk, H), jnp.float32),
            pltpu.SemaphoreType.DMA,
        ],
    )
    def k(table_hbm, idx_hbm, out_hbm, idx_v, rows_v, sem):
        wid = lax.axis_index("s") * _NC + lax.axis_index("c")
        my_idx = idx_hbm.at[wid]

        def body(c, _):
            pltpu.sync_copy(my_idx.at[pl.ds(c * 8, 8)], idx_v)
            cps = [
                pltpu.async_copy(table_hbm.at[idx_v.at[j]],
                                 rows_v.at[pl.ds(j * 125, 125)], sem)
                for j in range(8)
            ]
            for cp in cps:
                cp.wait()
            pltpu.sync_copy(rows_v,
                            out_hbm.at[pl.ds(wid * per_w + c * chunk, chunk)])
            return 0

        lax.fori_loop(0, n_chunks, body, 0)

    return k(table, idx3d)


@functools.partial(jax.jit, static_argnames=("d",))
def _sc_scatter(msgs, dst3d, zeros, d):
    per_w = E // _NW
    chunk = 1000
    n_chunks = per_w // chunk
    mesh = plsc.VectorSubcoreMesh(core_axis_name="c", subcore_axis_name="s")

    @functools.partial(
        pl.kernel, mesh=mesh,
        compiler_params=pltpu.CompilerParams(use_tc_tiling_on_sc=False),
        out_type=jax.ShapeDtypeStruct((_NC, N, d), jnp.float32),
        scratch_types=[
            pltpu.VMEM((8, 125), jnp.int32),
            pltpu.VMEM((chunk, d), jnp.float32),
            pltpu.SemaphoreType.DMA,
            pltpu.VMEM_SHARED((N, d), jnp.float32),
        ],
    )
    def k(msg_hbm, dst_hbm, zero_hbm, out_hbm, idx_v, m_v, sem, acc):
        cid = lax.axis_index("c")
        sid = lax.axis_index("s")
        wid = sid * _NC + cid
        base = wid * per_w
        my_dst = dst_hbm.at[wid]

        @pl.when(sid < 15)
        def _():
            s = pl.ds(sid * 640, 640)
            pltpu.sync_copy(zero_hbm.at[s], acc.at[s])

        @pl.when(sid == 15)
        def _():
            s = pl.ds(9600, 400)
            pltpu.sync_copy(zero_hbm.at[s], acc.at[s])

        plsc.subcore_barrier()

        def body(c, _):
            pltpu.sync_copy(my_dst.at[pl.ds(c * 8, 8)], idx_v)
            pltpu.sync_copy(msg_hbm.at[pl.ds(base + c * chunk, chunk)], m_v)
            for j in range(8):
                pltpu.sync_copy(m_v.at[pl.ds(j * 125, 125)],
                                acc.at[idx_v.at[j]], add=True)
            return 0

        lax.fori_loop(0, n_chunks, body, 0)

        plsc.subcore_barrier()
        @pl.when(sid < 15)
        def _():
            s = pl.ds(sid * 640, 640)
            pltpu.sync_copy(acc.at[s], out_hbm.at[cid].at[s])

        @pl.when(sid == 15)
        def _():
            s = pl.ds(9600, 400)
            pltpu.sync_copy(acc.at[s], out_hbm.at[cid].at[s])

    return k(msgs, dst3d, zeros)


_BN = 1000
_BE = 2000


def _full(shape):
    nd = len(shape)
    return pl.BlockSpec(shape, lambda i, _nd=nd: (0,) * _nd)


def _rows(shape, bs):
    blk = (bs,) + shape[1:]
    nd = len(shape)
    return pl.BlockSpec(blk, lambda i, _nd=nd: (i,) + (0,) * (_nd - 1))


def _tc_call(body, grid, in_arrays, in_row_flags, out_shapes, out_bs):
    in_specs = [
        _rows(a.shape, bs) if bs else _full(a.shape)
        for a, bs in zip(in_arrays, in_row_flags)
    ]
    out_specs = [_rows(s.shape, out_bs) for s in out_shapes]
    return pl.pallas_call(
        body, grid=grid, in_specs=in_specs, out_specs=out_specs,
        out_shape=out_shapes,
    )(*in_arrays)


def _node_enc(x, w1e, w2e, w1h, w2h):
    def body(x_r, w1e_r, w2e_r, w1h_r, w2h_r, oe_r, oh_r):
        xb = x_r[...]
        oe_r[...] = jnp.maximum(_mm(jnp.maximum(_mm(xb, w1e_r[...]), 0.0),
                                    w2e_r[...]), 0.0)
        oh_r[...] = jnp.maximum(_mm(jnp.maximum(_mm(xb, w1h_r[...]), 0.0),
                                    w2h_r[...]), 0.0)

    outs = [jax.ShapeDtypeStruct((N, H), jnp.float32)] * 2
    return _tc_call(body, (N // _BN,), [x, w1e, w2e, w1h, w2h],
                    [_BN, 0, 0, 0, 0], outs, _BN)


def _edge_enc(ea, w1e, w2e, w1h, w2h):
    def body(ea_r, w1e_r, w2e_r, w1h_r, w2h_r, oe_r, oh_r):
        eb = ea_r[...]
        oe_r[...] = jnp.maximum(_mm(jnp.maximum(_mm(eb, w1e_r[...]), 0.0),
                                    w2e_r[...]), 0.0)
        oh_r[...] = jnp.maximum(_mm(jnp.maximum(_mm(eb, w1h_r[...]), 0.0),
                                    w2h_r[...]), 0.0)

    outs = [jax.ShapeDtypeStruct((E, 64), jnp.float32)] * 2
    return _tc_call(body, (E // _BE,), [ea, w1e, w2e, w1h, w2h],
                    [_BE, 0, 0, 0, 0], outs, _BE)


def _edge_rel(g, e_prev, wg, we, b1, w2, b2, mask=None):
    use_mask = mask is not None

    def body(*refs):
        if use_mask:
            (g_r, e_r, mk_r, wg_r, we_r, b1_r, w2_r, b2_r, me_r, en_r) = refs
        else:
            (g_r, e_r, wg_r, we_r, b1_r, w2_r, b2_r, me_r, en_r) = refs
        ec = _mm(e_r[...], we_r[...])
        if use_mask:
            ec = ec * mk_r[...]
        hidden = jnp.maximum(_mm(g_r[...], wg_r[...]) + ec + b1_r[...][None, :],
                             0.0)
        m = _mm(hidden, w2_r[...]) + b2_r[...][None, :]
        me_r[...] = m * mk_r[...] if use_mask else m
        en_r[...] = jnp.maximum(m, 0.0)

    arrays = [g, e_prev] + ([mask] if use_mask else []) + [wg, we, b1, w2, b2]
    flags = [_BE, _BE] + ([_BE] if use_mask else []) + [0, 0, 0, 0, 0]
    outs = [jax.ShapeDtypeStruct((E, 64), jnp.float32)] * 2
    return _tc_call(body, (E // _BE,), arrays, flags, outs, _BE)


def _node_obj(x, parts, wox, woa, bo1, wo2, bo2):
    def body(x_r, p_r, wox_r, woa_r, bo1_r, wo2_r, bo2_r, o_r):
        xb = x_r[...]
        agg = p_r[0] + p_r[1]
        hidden = jnp.maximum(
            _mm(xb, wox_r[...]) + _mm(agg, woa_r[...]) + bo1_r[...][None, :],
            0.0)
        xn = _mm(hidden, wo2_r[...]) + bo2_r[...][None, :]
        o_r[...] = ALPHA * xb + (1.0 - ALPHA) * jnp.maximum(xn, 0.0)

    def pspec():
        return pl.BlockSpec((2, _BN, 64), lambda i: (0, i, 0))

    in_specs = [_rows(x.shape, _BN), pspec(), _full(wox.shape),
                _full(woa.shape), _full(bo1.shape), _full(wo2.shape),
                _full(bo2.shape)]
    out_shape = jax.ShapeDtypeStruct((N, H), jnp.float32)
    return pl.pallas_call(
        body, grid=(N // _BN,), in_specs=in_specs,
        out_specs=_rows((N, H), _BN), out_shape=out_shape,
    )(x, parts, wox, woa, bo1, wo2, bo2)


def _w_mlp(e0, e1, e2, e3, w1, b1, w2, b2, w3, b3):
    def body(e0_r, e1_r, e2_r, e3_r, w1_r, b1_r, w2_r, b2_r, w3_r, b3_r,
             ew_r, mk_r):
        h1 = (_mm(e0_r[...], w1_r[0:64]) + _mm(e1_r[...], w1_r[64:128]) +
              _mm(e2_r[...], w1_r[128:192]) + _mm(e3_r[...], w1_r[192:256]) +
              b1_r[...][None, :])
        h1 = jnp.maximum(h1, 0.0)
        h2 = jnp.maximum(_mm(h1, w2_r[...]) + b2_r[...][None, :], 0.0)
        logit = _mm(h2, w3_r[...]) + b3_r[...][None, :]
        ew_r[...] = jax.nn.sigmoid(logit)
        mk_r[...] = (logit > 0.0).astype(jnp.float32)

    arrays = [e0, e1, e2, e3, w1, b1, w2, b2, w3, b3]
    flags = [_BE, _BE, _BE, _BE, 0, 0, 0, 0, 0, 0]
    outs = [jax.ShapeDtypeStruct((E, 1), jnp.float32)] * 2
    return _tc_call(body, (E // _BE,), arrays, flags, outs, _BE)


def _p_rel(g, e0, e1, e2, e3, mask, wg, we, b1, w2, b2):
    def body(g_r, e0_r, e1_r, e2_r, e3_r, mk_r, wg_r, we_r, b1_r, w2_r, b2_r,
             o_r):
        ec = (_mm(e0_r[...], we_r[0:64]) + _mm(e1_r[...], we_r[64:128]) +
              _mm(e2_r[...], we_r[128:192]) + _mm(e3_r[...], we_r[192:256]))
        mk = mk_r[...]
        hidden = jnp.maximum(
            _mm(g_r[...], wg_r[...]) + ec * mk + b1_r[...][None, :], 0.0)
        m = (_mm(hidden, w2_r[...]) + b2_r[...][None, :]) * mk
        o_r[...] = jnp.concatenate(
            [m, jnp.zeros((m.shape[0], 15), jnp.float32)], axis=1)

    arrays = [g, e0, e1, e2, e3, mask, wg, we, b1, w2, b2]
    flags = [_BE, _BE, _BE, _BE, _BE, _BE, 0, 0, 0, 0, 0]
    outs = [jax.ShapeDtypeStruct((E, 16), jnp.float32)]
    return _tc_call(body, (E // _BE,), arrays, flags, outs, _BE)[0]


def _final_node(h, parts, bw1, bb1, bw2, bb2, bw3, bb3,
                xw1, xb1, xw2, xb2, xw3, xb3,
                pwh, pwa, pb1, pw2, pb2):
    def body(h_r, p_r, bw1_r, bb1_r, bw2_r, bb2_r, bw3_r, bb3_r,
             xw1_r, xb1_r, xw2_r, xb2_r, xw3_r, xb3_r,
             pwh_r, pwa_r, pb1_r, pw2_r, pb2_r,
             beta_r, hout_r, trk_r):
        hb = h_r[...]
        agg = (p_r[0] + p_r[1])[:, 0:1]
        b1h = jnp.maximum(_mm(hb, bw1_r[...]) + bb1_r[...][None, :], 0.0)
        b2h = jnp.maximum(_mm(b1h, bw2_r[...]) + bb2_r[...][None, :], 0.0)
        beta_r[...] = jax.nn.sigmoid(_mm(b2h, bw3_r[...]) +
                                     bb3_r[...][None, :]) + 1e-08
        x1h = jnp.maximum(_mm(hb, xw1_r[...]) + xb1_r[...][None, :], 0.0)
        x2h = jnp.maximum(_mm(x1h, xw2_r[...]) + xb2_r[...][None, :], 0.0)
        hout_r[...] = _mm(x2h, xw3_r[...]) + xb3_r[...][None, :]
        p1h = jnp.maximum(_mm(hb, pwh_r[...]) + agg * pwa_r[...][None, :] +
                          pb1_r[...][None, :], 0.0)
        trk_r[...] = _mm(p1h, pw2_r[...]) + pb2_r[...][None, :]

    def pspec():
        return pl.BlockSpec((2, _BN, 16), lambda i: (0, i, 0))

    weights = [bw1, bb1, bw2, bb2, bw3, bb3, xw1, xb1, xw2, xb2, xw3, xb3,
               pwh, pwa, pb1, pw2, pb2]
    in_specs = [_rows(h.shape, _BN), pspec()] + [_full(w.shape) for w in weights]
    outs = [jax.ShapeDtypeStruct((N, 1), jnp.float32),
            jax.ShapeDtypeStruct((N, 24), jnp.float32),
            jax.ShapeDtypeStruct((N, 1), jnp.float32)]
    out_specs = [_rows(o.shape, _BN) for o in outs]
    return pl.pallas_call(
        body, grid=(N // _BN,), in_specs=in_specs, out_specs=out_specs,
        out_shape=outs,
    )(h, parts, *weights)


def _resin(h, e_raw, idx2d, dst2d, zeros64, layer_params, mask=None):
    e_list = [e_raw]
    for p in layer_params:
        wr1, br1 = p["rel"][0]
        wr2, br2 = p["rel"][1]
        wo1, bo1 = p["obj"][0]
        wo2, bo2 = p["obj"][1]
        g = _sc_gather(h, idx2d, 2 * E).reshape(E, 2 * H)
        m_eff, e_next = _edge_rel(g, e_list[-1], wr1[0:2 * H], wr1[2 * H:],
                                  br1, wr2, br2, mask=mask)
        parts = _sc_scatter(m_eff, dst2d, zeros64, 64)
        h = _node_obj(h, parts, wo1[0:H], wo1[H:], bo1, wo2, bo2)
        e_list.append(e_next)
    return h, e_list


def kernel(x, edge_index, edge_attr, params):
    idx2d = edge_index.T.reshape(_NW, 80, 125)
    dst2d = edge_index[1].reshape(_NW, 40, 125)
    zeros64 = jnp.zeros((N, 64), jnp.float32)
    zeros16 = jnp.zeros((N, 16), jnp.float32)

    h_ec, h_hc = _node_enc(x, params["ec_node_enc"][0][0],
                           params["ec_node_enc"][1][0],
                           params["hc_node_enc"][0][0],
                           params["hc_node_enc"][1][0])
    e_ec, e_hc = _edge_enc(edge_attr, params["ec_edge_enc"][0][0],
                           params["ec_edge_enc"][1][0],
                           params["hc_edge_enc"][0][0],
                           params["hc_edge_enc"][1][0])

    h_ec, e_list_ec = _resin(h_ec, e_ec, idx2d, dst2d, zeros64,
                             params["ec_resin"])
    (ww1, wb1), (ww2, wb2), (ww3, wb3) = params["W_mlp"]
    edge_weights, mask = _w_mlp(*e_list_ec, ww1, wb1, ww2, wb2, ww3, wb3)

    h_hc, e_list_hc = _resin(h_hc, e_hc, idx2d, dst2d, zeros64,
                             params["hc_resin"], mask=mask)

    pr1, prb1 = params["P_in"]["rel"][0]
    pr2, prb2 = params["P_in"]["rel"][1]
    g_p = _sc_gather(h_hc, idx2d, 2 * E).reshape(E, 2 * H)
    m_p = _p_rel(g_p, *e_list_hc, mask, pr1[0:2 * H], pr1[2 * H:],
                 prb1, pr2, prb2)
    p_parts = _sc_scatter(m_p, dst2d, zeros16, 16)

    (bw1, bb1), (bw2, bb2), (bw3, bb3) = params["B_mlp"]
    (xw1, xb1), (xw2, xb2), (xw3, xb3) = params["X_mlp"]
    po1, pob1 = params["P_in"]["obj"][0]
    po2, pob2 = params["P_in"]["obj"][1]
    beta, h_out, track = _final_node(
        h_hc, p_parts, bw1, bb1, bw2, bb2, bw3, bb3,
        xw1, xb1, xw2, xb2, xw3, xb3,
        po1[0:H], po1[H], pob1, po2, pob2)

    return edge_weights, h_out, beta, track


# speedup vs baseline: 1.7171x; 1.0016x over previous
import functools

import jax
import jax.numpy as jnp
from jax import lax
from jax.experimental import pallas as pl
from jax.experimental.pallas import tpu as pltpu
from jax.experimental.pallas import tpu_sc as plsc

N = 10000
E = 160000
H = 128
ALPHA = 0.5

_NC = 2
_NS = 16
_NW = _NC * _NS


def _mm(a, b):
    return jax.lax.dot_general(a, b, (((1,), (0,)), ((), ())),
                               preferred_element_type=jnp.float32)


@functools.partial(jax.jit, static_argnames=("ni",))
def _sc_gather(table, idx3d, ni):
    per_w = ni // _NW
    chunk = 1000
    n_chunks = per_w // chunk
    mesh = plsc.VectorSubcoreMesh(core_axis_name="c", subcore_axis_name="s")

    @functools.partial(
        pl.kernel, mesh=mesh,
        compiler_params=pltpu.CompilerParams(use_tc_tiling_on_sc=False),
        out_type=jax.ShapeDtypeStruct((ni, H), jnp.float32),
        scratch_types=[
            pltpu.VMEM((8, 125), jnp.int32),
            pltpu.VMEM((chunk, H), jnp.float32),
            pltpu.SemaphoreType.DMA,
        ],
    )
    def k(table_hbm, idx_hbm, out_hbm, idx_v, rows_v, sem):
        wid = lax.axis_index("s") * _NC + lax.axis_index("c")
        my_idx = idx_hbm.at[wid]

        def body(c, _):
            pltpu.sync_copy(my_idx.at[pl.ds(c * 8, 8)], idx_v)
            cps = [
                pltpu.async_copy(table_hbm.at[idx_v.at[j]],
                                 rows_v.at[pl.ds(j * 125, 125)], sem)
                for j in range(8)
            ]
            for cp in cps:
                cp.wait()
            pltpu.sync_copy(rows_v,
                            out_hbm.at[pl.ds(wid * per_w + c * chunk, chunk)])
            return 0

        lax.fori_loop(0, n_chunks, body, 0)

    return k(table, idx3d)


@functools.partial(jax.jit, static_argnames=("d",))
def _sc_scatter(msgs, dst3d, zeros, d):
    per_w = E // _NW
    chunk = 1000
    n_chunks = per_w // chunk
    mesh = plsc.VectorSubcoreMesh(core_axis_name="c", subcore_axis_name="s")

    @functools.partial(
        pl.kernel, mesh=mesh,
        compiler_params=pltpu.CompilerParams(use_tc_tiling_on_sc=False),
        out_type=jax.ShapeDtypeStruct((_NC, N, d), jnp.float32),
        scratch_types=[
            pltpu.VMEM((8, 125), jnp.int32),
            pltpu.VMEM((chunk, d), jnp.float32),
            pltpu.SemaphoreType.DMA,
            pltpu.VMEM_SHARED((N, d), jnp.float32),
        ],
    )
    def k(msg_hbm, dst_hbm, zero_hbm, out_hbm, idx_v, m_v, sem, acc):
        cid = lax.axis_index("c")
        sid = lax.axis_index("s")
        wid = sid * _NC + cid
        base = wid * per_w
        my_dst = dst_hbm.at[wid]

        @pl.when(sid < 15)
        def _():
            s = pl.ds(sid * 640, 640)
            pltpu.sync_copy(zero_hbm.at[s], acc.at[s])

        @pl.when(sid == 15)
        def _():
            s = pl.ds(9600, 400)
            pltpu.sync_copy(zero_hbm.at[s], acc.at[s])

        plsc.subcore_barrier()

        def body(c, _):
            pltpu.sync_copy(my_dst.at[pl.ds(c * 8, 8)], idx_v)
            pltpu.sync_copy(msg_hbm.at[pl.ds(base + c * chunk, chunk)], m_v)
            for j in range(8):
                pltpu.sync_copy(m_v.at[pl.ds(j * 125, 125)],
                                acc.at[idx_v.at[j]], add=True)
            return 0

        lax.fori_loop(0, n_chunks, body, 0)

        plsc.subcore_barrier()
        @pl.when(sid < 15)
        def _():
            s = pl.ds(sid * 640, 640)
            pltpu.sync_copy(acc.at[s], out_hbm.at[cid].at[s])

        @pl.when(sid == 15)
        def _():
            s = pl.ds(9600, 400)
            pltpu.sync_copy(acc.at[s], out_hbm.at[cid].at[s])

    return k(msgs, dst3d, zeros)


_BN = 1000
_BE = 2000


def _full(shape):
    nd = len(shape)
    return pl.BlockSpec(shape, lambda i, _nd=nd: (0,) * _nd)


def _rows(shape, bs):
    blk = (bs,) + shape[1:]
    nd = len(shape)
    return pl.BlockSpec(blk, lambda i, _nd=nd: (i,) + (0,) * (_nd - 1))


def _tc_call(body, grid, in_arrays, in_row_flags, out_shapes, out_bs):
    in_specs = [
        _rows(a.shape, bs) if bs else _full(a.shape)
        for a, bs in zip(in_arrays, in_row_flags)
    ]
    out_specs = [_rows(s.shape, out_bs) for s in out_shapes]
    return pl.pallas_call(
        body, grid=grid, in_specs=in_specs, out_specs=out_specs,
        out_shape=out_shapes,
    )(*in_arrays)


def _node_enc(x, w1e, w2e, w1h, w2h):
    def body(x_r, w1e_r, w2e_r, w1h_r, w2h_r, oe_r, oh_r):
        xb = x_r[...]
        oe_r[...] = jnp.maximum(_mm(jnp.maximum(_mm(xb, w1e_r[...]), 0.0),
                                    w2e_r[...]), 0.0)
        oh_r[...] = jnp.maximum(_mm(jnp.maximum(_mm(xb, w1h_r[...]), 0.0),
                                    w2h_r[...]), 0.0)

    outs = [jax.ShapeDtypeStruct((N, H), jnp.float32)] * 2
    return _tc_call(body, (N // _BN,), [x, w1e, w2e, w1h, w2h],
                    [_BN, 0, 0, 0, 0], outs, _BN)


def _edge_enc(ea, w1e, w2e, w1h, w2h):
    def body(ea_r, w1e_r, w2e_r, w1h_r, w2h_r, oe_r, oh_r):
        eb = ea_r[...]
        oe_r[...] = jnp.maximum(_mm(jnp.maximum(_mm(eb, w1e_r[...]), 0.0),
                                    w2e_r[...]), 0.0)
        oh_r[...] = jnp.maximum(_mm(jnp.maximum(_mm(eb, w1h_r[...]), 0.0),
                                    w2h_r[...]), 0.0)

    outs = [jax.ShapeDtypeStruct((E, 64), jnp.float32)] * 2
    return _tc_call(body, (E // _BE,), [ea, w1e, w2e, w1h, w2h],
                    [_BE, 0, 0, 0, 0], outs, _BE)


def _edge_rel(g, e_prev, wg, we, b1, w2, b2, mask=None):
    use_mask = mask is not None

    def body(*refs):
        if use_mask:
            (g_r, e_r, mk_r, wg_r, we_r, b1_r, w2_r, b2_r, me_r, en_r) = refs
        else:
            (g_r, e_r, wg_r, we_r, b1_r, w2_r, b2_r, me_r, en_r) = refs
        ec = _mm(e_r[...], we_r[...])
        if use_mask:
            ec = ec * mk_r[...]
        hidden = jnp.maximum(_mm(g_r[...], wg_r[...]) + ec + b1_r[...][None, :],
                             0.0)
        m = _mm(hidden, w2_r[...]) + b2_r[...][None, :]
        me_r[...] = m * mk_r[...] if use_mask else m
        en_r[...] = jnp.maximum(m, 0.0)

    arrays = [g, e_prev] + ([mask] if use_mask else []) + [wg, we, b1, w2, b2]
    flags = [_BE, _BE] + ([_BE] if use_mask else []) + [0, 0, 0, 0, 0]
    outs = [jax.ShapeDtypeStruct((E, 64), jnp.float32)] * 2
    return _tc_call(body, (E // _BE,), arrays, flags, outs, _BE)


def _edge_rel_e_only(g, e_prev, wg, we, b1, w2, b2):
    def body(g_r, e_r, wg_r, we_r, b1_r, w2_r, b2_r, en_r):
        hidden = jnp.maximum(
            _mm(g_r[...], wg_r[...]) + _mm(e_r[...], we_r[...]) +
            b1_r[...][None, :], 0.0)
        m = _mm(hidden, w2_r[...]) + b2_r[...][None, :]
        en_r[...] = jnp.maximum(m, 0.0)

    arrays = [g, e_prev, wg, we, b1, w2, b2]
    flags = [_BE, _BE, 0, 0, 0, 0, 0]
    outs = [jax.ShapeDtypeStruct((E, 64), jnp.float32)]
    return _tc_call(body, (E // _BE,), arrays, flags, outs, _BE)[0]


def _node_obj(x, parts, wox, woa, bo1, wo2, bo2):
    def body(x_r, p_r, wox_r, woa_r, bo1_r, wo2_r, bo2_r, o_r):
        xb = x_r[...]
        agg = p_r[0] + p_r[1]
        hidden = jnp.maximum(
            _mm(xb, wox_r[...]) + _mm(agg, woa_r[...]) + bo1_r[...][None, :],
            0.0)
        xn = _mm(hidden, wo2_r[...]) + bo2_r[...][None, :]
        o_r[...] = ALPHA * xb + (1.0 - ALPHA) * jnp.maximum(xn, 0.0)

    def pspec():
        return pl.BlockSpec((2, _BN, 64), lambda i: (0, i, 0))

    in_specs = [_rows(x.shape, _BN), pspec(), _full(wox.shape),
                _full(woa.shape), _full(bo1.shape), _full(wo2.shape),
                _full(bo2.shape)]
    out_shape = jax.ShapeDtypeStruct((N, H), jnp.float32)
    return pl.pallas_call(
        body, grid=(N // _BN,), in_specs=in_specs,
        out_specs=_rows((N, H), _BN), out_shape=out_shape,
    )(x, parts, wox, woa, bo1, wo2, bo2)


def _w_mlp(e0, e1, e2, e3, w1, b1, w2, b2, w3, b3):
    def body(e0_r, e1_r, e2_r, e3_r, w1_r, b1_r, w2_r, b2_r, w3_r, b3_r,
             ew_r, mk_r):
        h1 = (_mm(e0_r[...], w1_r[0:64]) + _mm(e1_r[...], w1_r[64:128]) +
              _mm(e2_r[...], w1_r[128:192]) + _mm(e3_r[...], w1_r[192:256]) +
              b1_r[...][None, :])
        h1 = jnp.maximum(h1, 0.0)
        h2 = jnp.maximum(_mm(h1, w2_r[...]) + b2_r[...][None, :], 0.0)
        logit = _mm(h2, w3_r[...]) + b3_r[...][None, :]
        ew_r[...] = jax.nn.sigmoid(logit)
        mk_r[...] = (logit > 0.0).astype(jnp.float32)

    arrays = [e0, e1, e2, e3, w1, b1, w2, b2, w3, b3]
    flags = [_BE, _BE, _BE, _BE, 0, 0, 0, 0, 0, 0]
    outs = [jax.ShapeDtypeStruct((E, 1), jnp.float32)] * 2
    return _tc_call(body, (E // _BE,), arrays, flags, outs, _BE)


def _p_rel(g, e0, e1, e2, e3, mask, wg, we, b1, w2, b2):
    def body(g_r, e0_r, e1_r, e2_r, e3_r, mk_r, wg_r, we_r, b1_r, w2_r, b2_r,
             o_r):
        ec = (_mm(e0_r[...], we_r[0:64]) + _mm(e1_r[...], we_r[64:128]) +
              _mm(e2_r[...], we_r[128:192]) + _mm(e3_r[...], we_r[192:256]))
        mk = mk_r[...]
        hidden = jnp.maximum(
            _mm(g_r[...], wg_r[...]) + ec * mk + b1_r[...][None, :], 0.0)
        m = (_mm(hidden, w2_r[...]) + b2_r[...][None, :]) * mk
        o_r[...] = jnp.concatenate(
            [m, jnp.zeros((m.shape[0], 15), jnp.float32)], axis=1)

    arrays = [g, e0, e1, e2, e3, mask, wg, we, b1, w2, b2]
    flags = [_BE, _BE, _BE, _BE, _BE, _BE, 0, 0, 0, 0, 0]
    outs = [jax.ShapeDtypeStruct((E, 16), jnp.float32)]
    return _tc_call(body, (E // _BE,), arrays, flags, outs, _BE)[0]


def _final_node(h, parts, bw1, bb1, bw2, bb2, bw3, bb3,
                xw1, xb1, xw2, xb2, xw3, xb3,
                pwh, pwa, pb1, pw2, pb2):
    def body(h_r, p_r, bw1_r, bb1_r, bw2_r, bb2_r, bw3_r, bb3_r,
             xw1_r, xb1_r, xw2_r, xb2_r, xw3_r, xb3_r,
             pwh_r, pwa_r, pb1_r, pw2_r, pb2_r,
             beta_r, hout_r, trk_r):
        hb = h_r[...]
        agg = (p_r[0] + p_r[1])[:, 0:1]
        b1h = jnp.maximum(_mm(hb, bw1_r[...]) + bb1_r[...][None, :], 0.0)
        b2h = jnp.maximum(_mm(b1h, bw2_r[...]) + bb2_r[...][None, :], 0.0)
        beta_r[...] = jax.nn.sigmoid(_mm(b2h, bw3_r[...]) +
                                     bb3_r[...][None, :]) + 1e-08
        x1h = jnp.maximum(_mm(hb, xw1_r[...]) + xb1_r[...][None, :], 0.0)
        x2h = jnp.maximum(_mm(x1h, xw2_r[...]) + xb2_r[...][None, :], 0.0)
        hout_r[...] = _mm(x2h, xw3_r[...]) + xb3_r[...][None, :]
        p1h = jnp.maximum(_mm(hb, pwh_r[...]) + agg * pwa_r[...][None, :] +
                          pb1_r[...][None, :], 0.0)
        trk_r[...] = _mm(p1h, pw2_r[...]) + pb2_r[...][None, :]

    def pspec():
        return pl.BlockSpec((2, _BN, 16), lambda i: (0, i, 0))

    weights = [bw1, bb1, bw2, bb2, bw3, bb3, xw1, xb1, xw2, xb2, xw3, xb3,
               pwh, pwa, pb1, pw2, pb2]
    in_specs = [_rows(h.shape, _BN), pspec()] + [_full(w.shape) for w in weights]
    outs = [jax.ShapeDtypeStruct((N, 1), jnp.float32),
            jax.ShapeDtypeStruct((N, 24), jnp.float32),
            jax.ShapeDtypeStruct((N, 1), jnp.float32)]
    out_specs = [_rows(o.shape, _BN) for o in outs]
    return pl.pallas_call(
        body, grid=(N // _BN,), in_specs=in_specs, out_specs=out_specs,
        out_shape=outs,
    )(h, parts, *weights)


def _resin(h, e_raw, idx2d, dst2d, zeros64, layer_params, mask=None,
           skip_last_node=False):
    e_list = [e_raw]
    last = len(layer_params) - 1
    for li, p in enumerate(layer_params):
        wr1, br1 = p["rel"][0]
        wr2, br2 = p["rel"][1]
        g = _sc_gather(h, idx2d, 2 * E).reshape(E, 2 * H)
        if skip_last_node and li == last:
            e_next = _edge_rel_e_only(g, e_list[-1], wr1[0:2 * H],
                                      wr1[2 * H:], br1, wr2, br2)
            e_list.append(e_next)
            break
        wo1, bo1 = p["obj"][0]
        wo2, bo2 = p["obj"][1]
        m_eff, e_next = _edge_rel(g, e_list[-1], wr1[0:2 * H], wr1[2 * H:],
                                  br1, wr2, br2, mask=mask)
        parts = _sc_scatter(m_eff, dst2d, zeros64, 64)
        h = _node_obj(h, parts, wo1[0:H], wo1[H:], bo1, wo2, bo2)
        e_list.append(e_next)
    return h, e_list


def kernel(x, edge_index, edge_attr, params):
    idx2d = edge_index.T.reshape(_NW, 80, 125)
    dst2d = edge_index[1].reshape(_NW, 40, 125)
    zeros64 = jnp.zeros((N, 64), jnp.float32)
    zeros16 = jnp.zeros((N, 16), jnp.float32)

    h_ec, h_hc = _node_enc(x, params["ec_node_enc"][0][0],
                           params["ec_node_enc"][1][0],
                           params["hc_node_enc"][0][0],
                           params["hc_node_enc"][1][0])
    e_ec, e_hc = _edge_enc(edge_attr, params["ec_edge_enc"][0][0],
                           params["ec_edge_enc"][1][0],
                           params["hc_edge_enc"][0][0],
                           params["hc_edge_enc"][1][0])

    _, e_list_ec = _resin(h_ec, e_ec, idx2d, dst2d, zeros64,
                          params["ec_resin"], skip_last_node=True)
    (ww1, wb1), (ww2, wb2), (ww3, wb3) = params["W_mlp"]
    edge_weights, mask = _w_mlp(*e_list_ec, ww1, wb1, ww2, wb2, ww3, wb3)

    h_hc, e_list_hc = _resin(h_hc, e_hc, idx2d, dst2d, zeros64,
                             params["hc_resin"], mask=mask)

    pr1, prb1 = params["P_in"]["rel"][0]
    pr2, prb2 = params["P_in"]["rel"][1]
    g_p = _sc_gather(h_hc, idx2d, 2 * E).reshape(E, 2 * H)
    m_p = _p_rel(g_p, *e_list_hc, mask, pr1[0:2 * H], pr1[2 * H:],
                 prb1, pr2, prb2)
    p_parts = _sc_scatter(m_p, dst2d, zeros16, 16)

    (bw1, bb1), (bw2, bb2), (bw3, bb3) = params["B_mlp"]
    (xw1, xb1), (xw2, xb2), (xw3, xb3) = params["X_mlp"]
    po1, pob1 = params["P_in"]["obj"][0]
    po2, pob2 = params["P_in"]["obj"][1]
    beta, h_out, track = _final_node(
        h_hc, p_parts, bw1, bb1, bw2, bb2, bw3, bb3,
        xw1, xb1, xw2, xb2, xw3, xb3,
        po1[0:H], po1[H], pob1, po2, pob2)

    return edge_weights, h_out, beta, track


# speedup vs baseline: 1.7191x; 1.0012x over previous
import functools

import jax
import jax.numpy as jnp
from jax import lax
from jax.experimental import pallas as pl
from jax.experimental.pallas import tpu as pltpu
from jax.experimental.pallas import tpu_sc as plsc

N = 10000
E = 160000
H = 128
ALPHA = 0.5

_NC = 2
_NS = 16
_NW = _NC * _NS


def _mm(a, b):
    return jax.lax.dot_general(a, b, (((1,), (0,)), ((), ())),
                               preferred_element_type=jnp.float32)


@functools.partial(jax.jit, static_argnames=("ni",))
def _sc_gather(table, idx3d, ni):
    per_w = ni // _NW
    chunk = 1000
    n_chunks = per_w // chunk
    mesh = plsc.VectorSubcoreMesh(core_axis_name="c", subcore_axis_name="s")

    @functools.partial(
        pl.kernel, mesh=mesh,
        compiler_params=pltpu.CompilerParams(use_tc_tiling_on_sc=False),
        out_type=jax.ShapeDtypeStruct((ni, H), jnp.float32),
        scratch_types=[
            pltpu.VMEM((8, 125), jnp.int32),
            pltpu.VMEM((chunk, H), jnp.float32),
            pltpu.SemaphoreType.DMA,
        ],
    )
    def k(table_hbm, idx_hbm, out_hbm, idx_v, rows_v, sem):
        wid = lax.axis_index("s") * _NC + lax.axis_index("c")
        my_idx = idx_hbm.at[wid]

        def body(c, _):
            pltpu.sync_copy(my_idx.at[pl.ds(c * 8, 8)], idx_v)
            cps = [
                pltpu.async_copy(table_hbm.at[idx_v.at[j]],
                                 rows_v.at[pl.ds(j * 125, 125)], sem)
                for j in range(8)
            ]
            for cp in cps:
                cp.wait()
            pltpu.sync_copy(rows_v,
                            out_hbm.at[pl.ds(wid * per_w + c * chunk, chunk)])
            return 0

        lax.fori_loop(0, n_chunks, body, 0)

    return k(table, idx3d)


@functools.partial(jax.jit, static_argnames=("d",))
def _sc_scatter(msgs, dst3d, zeros, d):
    per_w = E // _NW
    chunk = 1000
    n_chunks = per_w // chunk
    mesh = plsc.VectorSubcoreMesh(core_axis_name="c", subcore_axis_name="s")

    @functools.partial(
        pl.kernel, mesh=mesh,
        compiler_params=pltpu.CompilerParams(use_tc_tiling_on_sc=False),
        out_type=jax.ShapeDtypeStruct((_NC, N, d), jnp.float32),
        scratch_types=[
            pltpu.VMEM((8, 125), jnp.int32),
            pltpu.VMEM((chunk, d), jnp.float32),
            pltpu.SemaphoreType.DMA,
            pltpu.VMEM_SHARED((N, d), jnp.float32),
        ],
    )
    def k(msg_hbm, dst_hbm, zero_hbm, out_hbm, idx_v, m_v, sem, acc):
        cid = lax.axis_index("c")
        sid = lax.axis_index("s")
        wid = sid * _NC + cid
        base = wid * per_w
        my_dst = dst_hbm.at[wid]

        @pl.when(sid < 15)
        def _():
            s = pl.ds(sid * 640, 640)
            pltpu.sync_copy(zero_hbm.at[s], acc.at[s])

        @pl.when(sid == 15)
        def _():
            s = pl.ds(9600, 400)
            pltpu.sync_copy(zero_hbm.at[s], acc.at[s])

        plsc.subcore_barrier()

        def body(c, _):
            pltpu.sync_copy(my_dst.at[pl.ds(c * 8, 8)], idx_v)
            pltpu.sync_copy(msg_hbm.at[pl.ds(base + c * chunk, chunk)], m_v)
            for j in range(8):
                pltpu.sync_copy(m_v.at[pl.ds(j * 125, 125)],
                                acc.at[idx_v.at[j]], add=True)
            return 0

        lax.fori_loop(0, n_chunks, body, 0)

        plsc.subcore_barrier()
        @pl.when(sid < 15)
        def _():
            s = pl.ds(sid * 640, 640)
            pltpu.sync_copy(acc.at[s], out_hbm.at[cid].at[s])

        @pl.when(sid == 15)
        def _():
            s = pl.ds(9600, 400)
            pltpu.sync_copy(acc.at[s], out_hbm.at[cid].at[s])

    return k(msgs, dst3d, zeros)


_BN = 1000
_BE = 2000


def _full(shape):
    nd = len(shape)
    return pl.BlockSpec(shape, lambda i, _nd=nd: (0,) * _nd)


def _rows(shape, bs):
    blk = (bs,) + shape[1:]
    nd = len(shape)
    return pl.BlockSpec(blk, lambda i, _nd=nd: (i,) + (0,) * (_nd - 1))


def _tc_call(body, grid, in_arrays, in_row_flags, out_shapes, out_bs):
    in_specs = [
        _rows(a.shape, bs) if bs else _full(a.shape)
        for a, bs in zip(in_arrays, in_row_flags)
    ]
    out_specs = [_rows(s.shape, out_bs) for s in out_shapes]
    return pl.pallas_call(
        body, grid=grid, in_specs=in_specs, out_specs=out_specs,
        out_shape=out_shapes,
    )(*in_arrays)


def _node_enc(x, w1e, w2e, w1h, w2h):
    def body(x_r, w1e_r, w2e_r, w1h_r, w2h_r, oe_r, oh_r):
        xb = x_r[...]
        oe_r[...] = jnp.maximum(_mm(jnp.maximum(_mm(xb, w1e_r[...]), 0.0),
                                    w2e_r[...]), 0.0)
        oh_r[...] = jnp.maximum(_mm(jnp.maximum(_mm(xb, w1h_r[...]), 0.0),
                                    w2h_r[...]), 0.0)

    outs = [jax.ShapeDtypeStruct((N, H), jnp.float32)] * 2
    return _tc_call(body, (N // _BN,), [x, w1e, w2e, w1h, w2h],
                    [_BN, 0, 0, 0, 0], outs, _BN)


def _edge_enc(ea, w1e, w2e, w1h, w2h):
    def body(ea_r, w1e_r, w2e_r, w1h_r, w2h_r, oe_r, oh_r):
        eb = ea_r[...]
        oe_r[...] = jnp.maximum(_mm(jnp.maximum(_mm(eb, w1e_r[...]), 0.0),
                                    w2e_r[...]), 0.0)
        oh_r[...] = jnp.maximum(_mm(jnp.maximum(_mm(eb, w1h_r[...]), 0.0),
                                    w2h_r[...]), 0.0)

    outs = [jax.ShapeDtypeStruct((E, 64), jnp.float32)] * 2
    return _tc_call(body, (E // _BE,), [ea, w1e, w2e, w1h, w2h],
                    [_BE, 0, 0, 0, 0], outs, _BE)


def _edge_rel(g, e_prev, w1, b1, w2, b2, mask=None):
    use_mask = mask is not None

    def body(*refs):
        if use_mask:
            (g_r, e_r, mk_r, w1_r, b1_r, w2_r, b2_r, me_r, en_r) = refs
        else:
            (g_r, e_r, w1_r, b1_r, w2_r, b2_r, me_r, en_r) = refs
        eb = e_r[...]
        if use_mask:
            eb = eb * mk_r[...]
        cat = jnp.concatenate([g_r[...], eb], axis=1)
        hidden = jnp.maximum(_mm(cat, w1_r[...]) + b1_r[...][None, :], 0.0)
        m = _mm(hidden, w2_r[...]) + b2_r[...][None, :]
        me_r[...] = m * mk_r[...] if use_mask else m
        en_r[...] = jnp.maximum(m, 0.0)

    arrays = [g, e_prev] + ([mask] if use_mask else []) + [w1, b1, w2, b2]
    flags = [_BE, _BE] + ([_BE] if use_mask else []) + [0, 0, 0, 0]
    outs = [jax.ShapeDtypeStruct((E, 64), jnp.float32)] * 2
    return _tc_call(body, (E // _BE,), arrays, flags, outs, _BE)


def _edge_rel_e_only(g, e_prev, w1, b1, w2, b2):
    def body(g_r, e_r, w1_r, b1_r, w2_r, b2_r, en_r):
        cat = jnp.concatenate([g_r[...], e_r[...]], axis=1)
        hidden = jnp.maximum(_mm(cat, w1_r[...]) + b1_r[...][None, :], 0.0)
        m = _mm(hidden, w2_r[...]) + b2_r[...][None, :]
        en_r[...] = jnp.maximum(m, 0.0)

    arrays = [g, e_prev, w1, b1, w2, b2]
    flags = [_BE, _BE, 0, 0, 0, 0]
    outs = [jax.ShapeDtypeStruct((E, 64), jnp.float32)]
    return _tc_call(body, (E // _BE,), arrays, flags, outs, _BE)[0]


def _node_obj(x, parts, wo1, bo1, wo2, bo2):
    def body(x_r, p_r, wo1_r, bo1_r, wo2_r, bo2_r, o_r):
        xb = x_r[...]
        agg = p_r[0] + p_r[1]
        cat = jnp.concatenate([xb, agg], axis=1)
        hidden = jnp.maximum(_mm(cat, wo1_r[...]) + bo1_r[...][None, :], 0.0)
        xn = _mm(hidden, wo2_r[...]) + bo2_r[...][None, :]
        o_r[...] = ALPHA * xb + (1.0 - ALPHA) * jnp.maximum(xn, 0.0)

    def pspec():
        return pl.BlockSpec((2, _BN, 64), lambda i: (0, i, 0))

    in_specs = [_rows(x.shape, _BN), pspec(), _full(wo1.shape),
                _full(bo1.shape), _full(wo2.shape), _full(bo2.shape)]
    out_shape = jax.ShapeDtypeStruct((N, H), jnp.float32)
    return pl.pallas_call(
        body, grid=(N // _BN,), in_specs=in_specs,
        out_specs=_rows((N, H), _BN), out_shape=out_shape,
    )(x, parts, wo1, bo1, wo2, bo2)


def _w_mlp(e0, e1, e2, e3, w1, b1, w2, b2, w3, b3):
    def body(e0_r, e1_r, e2_r, e3_r, w1_r, b1_r, w2_r, b2_r, w3_r, b3_r,
             ew_r, mk_r):
        cat = jnp.concatenate([e0_r[...], e1_r[...], e2_r[...], e3_r[...]],
                              axis=1)
        h1 = jnp.maximum(_mm(cat, w1_r[...]) + b1_r[...][None, :], 0.0)
        h2 = jnp.maximum(_mm(h1, w2_r[...]) + b2_r[...][None, :], 0.0)
        logit = _mm(h2, w3_r[...]) + b3_r[...][None, :]
        ew = jax.nn.sigmoid(logit)
        ew_r[...] = ew
        mk_r[...] = (ew > 0.5).astype(jnp.float32)

    arrays = [e0, e1, e2, e3, w1, b1, w2, b2, w3, b3]
    flags = [_BE, _BE, _BE, _BE, 0, 0, 0, 0, 0, 0]
    outs = [jax.ShapeDtypeStruct((E, 1), jnp.float32)] * 2
    return _tc_call(body, (E // _BE,), arrays, flags, outs, _BE)


def _p_rel(g, e0, e1, e2, e3, mask, w1, b1, w2, b2):
    def body(g_r, e0_r, e1_r, e2_r, e3_r, mk_r, w1_r, b1_r, w2_r, b2_r,
             o_r):
        mk = mk_r[...]
        cat = jnp.concatenate(
            [g_r[...], e0_r[...] * mk, e1_r[...] * mk, e2_r[...] * mk,
             e3_r[...] * mk], axis=1)
        hidden = jnp.maximum(_mm(cat, w1_r[...]) + b1_r[...][None, :], 0.0)
        m = (_mm(hidden, w2_r[...]) + b2_r[...][None, :]) * mk
        o_r[...] = jnp.concatenate(
            [m, jnp.zeros((m.shape[0], 15), jnp.float32)], axis=1)

    arrays = [g, e0, e1, e2, e3, mask, w1, b1, w2, b2]
    flags = [_BE, _BE, _BE, _BE, _BE, _BE, 0, 0, 0, 0]
    outs = [jax.ShapeDtypeStruct((E, 16), jnp.float32)]
    return _tc_call(body, (E // _BE,), arrays, flags, outs, _BE)[0]


def _final_node(h, parts, bw1, bb1, bw2, bb2, bw3, bb3,
                xw1, xb1, xw2, xb2, xw3, xb3,
                pwh, pwa, pb1, pw2, pb2):
    def body(h_r, p_r, bw1_r, bb1_r, bw2_r, bb2_r, bw3_r, bb3_r,
             xw1_r, xb1_r, xw2_r, xb2_r, xw3_r, xb3_r,
             pwh_r, pwa_r, pb1_r, pw2_r, pb2_r,
             beta_r, hout_r, trk_r):
        hb = h_r[...]
        agg = (p_r[0] + p_r[1])[:, 0:1]
        b1h = jnp.maximum(_mm(hb, bw1_r[...]) + bb1_r[...][None, :], 0.0)
        b2h = jnp.maximum(_mm(b1h, bw2_r[...]) + bb2_r[...][None, :], 0.0)
        beta_r[...] = jax.nn.sigmoid(_mm(b2h, bw3_r[...]) +
                                     bb3_r[...][None, :]) + 1e-08
        x1h = jnp.maximum(_mm(hb, xw1_r[...]) + xb1_r[...][None, :], 0.0)
        x2h = jnp.maximum(_mm(x1h, xw2_r[...]) + xb2_r[...][None, :], 0.0)
        hout_r[...] = _mm(x2h, xw3_r[...]) + xb3_r[...][None, :]
        p1h = jnp.maximum(_mm(hb, pwh_r[...]) + agg * pwa_r[...][None, :] +
                          pb1_r[...][None, :], 0.0)
        trk_r[...] = _mm(p1h, pw2_r[...]) + pb2_r[...][None, :]

    def pspec():
        return pl.BlockSpec((2, _BN, 16), lambda i: (0, i, 0))

    weights = [bw1, bb1, bw2, bb2, bw3, bb3, xw1, xb1, xw2, xb2, xw3, xb3,
               pwh, pwa, pb1, pw2, pb2]
    in_specs = [_rows(h.shape, _BN), pspec()] + [_full(w.shape) for w in weights]
    outs = [jax.ShapeDtypeStruct((N, 1), jnp.float32),
            jax.ShapeDtypeStruct((N, 24), jnp.float32),
            jax.ShapeDtypeStruct((N, 1), jnp.float32)]
    out_specs = [_rows(o.shape, _BN) for o in outs]
    return pl.pallas_call(
        body, grid=(N // _BN,), in_specs=in_specs, out_specs=out_specs,
        out_shape=outs,
    )(h, parts, *weights)


def _resin(h, e_raw, idx2d, dst2d, zeros64, layer_params, mask=None,
           skip_last_node=False):
    e_list = [e_raw]
    last = len(layer_params) - 1
    for li, p in enumerate(layer_params):
        wr1, br1 = p["rel"][0]
        wr2, br2 = p["rel"][1]
        g = _sc_gather(h, idx2d, 2 * E).reshape(E, 2 * H)
        if skip_last_node and li == last:
            e_next = _edge_rel_e_only(g, e_list[-1], wr1, br1, wr2, br2)
            e_list.append(e_next)
            break
        wo1, bo1 = p["obj"][0]
        wo2, bo2 = p["obj"][1]
        m_eff, e_next = _edge_rel(g, e_list[-1], wr1, br1, wr2, br2,
                                  mask=mask)
        parts = _sc_scatter(m_eff, dst2d, zeros64, 64)
        h = _node_obj(h, parts, wo1, bo1, wo2, bo2)
        e_list.append(e_next)
    return h, e_list


def kernel(x, edge_index, edge_attr, params):
    idx2d = edge_index.T.reshape(_NW, 80, 125)
    dst2d = edge_index[1].reshape(_NW, 40, 125)
    zeros64 = jnp.zeros((N, 64), jnp.float32)
    zeros16 = jnp.zeros((N, 16), jnp.float32)

    h_ec, h_hc = _node_enc(x, params["ec_node_enc"][0][0],
                           params["ec_node_enc"][1][0],
                           params["hc_node_enc"][0][0],
                           params["hc_node_enc"][1][0])
    e_ec, e_hc = _edge_enc(edge_attr, params["ec_edge_enc"][0][0],
                           params["ec_edge_enc"][1][0],
                           params["hc_edge_enc"][0][0],
                           params["hc_edge_enc"][1][0])

    _, e_list_ec = _resin(h_ec, e_ec, idx2d, dst2d, zeros64,
                          params["ec_resin"], skip_last_node=True)
    (ww1, wb1), (ww2, wb2), (ww3, wb3) = params["W_mlp"]
    edge_weights, mask = _w_mlp(*e_list_ec, ww1, wb1, ww2, wb2, ww3, wb3)

    h_hc, e_list_hc = _resin(h_hc, e_hc, idx2d, dst2d, zeros64,
                             params["hc_resin"], mask=mask)

    pr1, prb1 = params["P_in"]["rel"][0]
    pr2, prb2 = params["P_in"]["rel"][1]
    g_p = _sc_gather(h_hc, idx2d, 2 * E).reshape(E, 2 * H)
    m_p = _p_rel(g_p, *e_list_hc, mask, pr1, prb1, pr2, prb2)
    p_parts = _sc_scatter(m_p, dst2d, zeros16, 16)

    (bw1, bb1), (bw2, bb2), (bw3, bb3) = params["B_mlp"]
    (xw1, xb1), (xw2, xb2), (xw3, xb3) = params["X_mlp"]
    po1, pob1 = params["P_in"]["obj"][0]
    po2, pob2 = params["P_in"]["obj"][1]
    beta, h_out, track = _final_node(
        h_hc, p_parts, bw1, bb1, bw2, bb2, bw3, bb3,
        xw1, xb1, xw2, xb2, xw3, xb3,
        po1[0:H], po1[H], pob1, po2, pob2)

    return edge_weights, h_out, beta, track


# speedup vs baseline: 1.7593x; 1.0234x over previous
import functools

import jax
import jax.numpy as jnp
from jax import lax
from jax.experimental import pallas as pl
from jax.experimental.pallas import tpu as pltpu
from jax.experimental.pallas import tpu_sc as plsc

N = 10000
E = 160000
H = 128
ALPHA = 0.5
EHS = (96000, 64000)

_NC = 2
_NS = 16
_NW = _NC * _NS


def _mm(a, b):
    return jax.lax.dot_general(a, b, (((1,), (0,)), ((), ())),
                               preferred_element_type=jnp.float32)


@functools.partial(jax.jit, static_argnames=("ni",))
def _sc_gather(table, idx3d, ni):
    per_w = ni // _NW
    chunk = 1000
    n_chunks = per_w // chunk
    mesh = plsc.VectorSubcoreMesh(core_axis_name="c", subcore_axis_name="s")

    @functools.partial(
        pl.kernel, mesh=mesh,
        compiler_params=pltpu.CompilerParams(use_tc_tiling_on_sc=False),
        out_type=jax.ShapeDtypeStruct((ni, H), jnp.float32),
        scratch_types=[
            pltpu.VMEM((8, 125), jnp.int32),
            pltpu.VMEM((chunk, H), jnp.float32),
            pltpu.SemaphoreType.DMA,
        ],
    )
    def k(table_hbm, idx_hbm, out_hbm, idx_v, rows_v, sem):
        wid = lax.axis_index("s") * _NC + lax.axis_index("c")
        my_idx = idx_hbm.at[wid]

        def body(c, _):
            pltpu.sync_copy(my_idx.at[pl.ds(c * 8, 8)], idx_v)
            cps = [
                pltpu.async_copy(table_hbm.at[idx_v.at[j]],
                                 rows_v.at[pl.ds(j * 125, 125)], sem)
                for j in range(8)
            ]
            for cp in cps:
                cp.wait()
            pltpu.sync_copy(rows_v,
                            out_hbm.at[pl.ds(wid * per_w + c * chunk, chunk)])
            return 0

        lax.fori_loop(0, n_chunks, body, 0)

    return k(table, idx3d)


@functools.partial(jax.jit, static_argnames=("d",))
def _sc_scatter(msgs, dst3d, zeros, d):
    ne = msgs.shape[0]
    per_w = ne // _NW
    chunk = 1000
    n_chunks = per_w // chunk
    mesh = plsc.VectorSubcoreMesh(core_axis_name="c", subcore_axis_name="s")

    @functools.partial(
        pl.kernel, mesh=mesh,
        compiler_params=pltpu.CompilerParams(use_tc_tiling_on_sc=False),
        out_type=jax.ShapeDtypeStruct((_NC, N, d), jnp.float32),
        scratch_types=[
            pltpu.VMEM((8, 125), jnp.int32),
            pltpu.VMEM((chunk, d), jnp.float32),
            pltpu.SemaphoreType.DMA,
            pltpu.VMEM_SHARED((N, d), jnp.float32),
        ],
    )
    def k(msg_hbm, dst_hbm, zero_hbm, out_hbm, idx_v, m_v, sem, acc):
        cid = lax.axis_index("c")
        sid = lax.axis_index("s")
        wid = sid * _NC + cid
        base = wid * per_w
        my_dst = dst_hbm.at[wid]

        @pl.when(sid < 15)
        def _():
            s = pl.ds(sid * 640, 640)
            pltpu.sync_copy(zero_hbm.at[s], acc.at[s])

        @pl.when(sid == 15)
        def _():
            s = pl.ds(9600, 400)
            pltpu.sync_copy(zero_hbm.at[s], acc.at[s])

        plsc.subcore_barrier()

        def body(c, _):
            pltpu.sync_copy(my_dst.at[pl.ds(c * 8, 8)], idx_v)
            pltpu.sync_copy(msg_hbm.at[pl.ds(base + c * chunk, chunk)], m_v)
            for j in range(8):
                pltpu.sync_copy(m_v.at[pl.ds(j * 125, 125)],
                                acc.at[idx_v.at[j]], add=True)
            return 0

        lax.fori_loop(0, n_chunks, body, 0)

        plsc.subcore_barrier()
        @pl.when(sid < 15)
        def _():
            s = pl.ds(sid * 640, 640)
            pltpu.sync_copy(acc.at[s], out_hbm.at[cid].at[s])

        @pl.when(sid == 15)
        def _():
            s = pl.ds(9600, 400)
            pltpu.sync_copy(acc.at[s], out_hbm.at[cid].at[s])

    return k(msgs, dst3d, zeros)


_BN = 1000
_BE = 2000


def _full(shape):
    nd = len(shape)
    return pl.BlockSpec(shape, lambda i, _nd=nd: (0,) * _nd)


def _rows(shape, bs):
    blk = (bs,) + shape[1:]
    nd = len(shape)
    return pl.BlockSpec(blk, lambda i, _nd=nd: (i,) + (0,) * (_nd - 1))


def _tc_call(body, grid, in_arrays, in_row_flags, out_shapes, out_bs):
    in_specs = [
        _rows(a.shape, bs) if bs else _full(a.shape)
        for a, bs in zip(in_arrays, in_row_flags)
    ]
    out_specs = [_rows(s.shape, out_bs) for s in out_shapes]
    return pl.pallas_call(
        body, grid=grid, in_specs=in_specs, out_specs=out_specs,
        out_shape=out_shapes,
    )(*in_arrays)


def _node_enc(x, w1e, w2e, w1h, w2h):
    def body(x_r, w1e_r, w2e_r, w1h_r, w2h_r, oe_r, oh_r):
        xb = x_r[...]
        oe_r[...] = jnp.maximum(_mm(jnp.maximum(_mm(xb, w1e_r[...]), 0.0),
                                    w2e_r[...]), 0.0)
        oh_r[...] = jnp.maximum(_mm(jnp.maximum(_mm(xb, w1h_r[...]), 0.0),
                                    w2h_r[...]), 0.0)

    outs = [jax.ShapeDtypeStruct((N, H), jnp.float32)] * 2
    return _tc_call(body, (N // _BN,), [x, w1e, w2e, w1h, w2h],
                    [_BN, 0, 0, 0, 0], outs, _BN)


def _edge_enc(ea, w1e, w2e, w1h, w2h):
    ne = ea.shape[0]

    def body(ea_r, w1e_r, w2e_r, w1h_r, w2h_r, oe_r, oh_r):
        eb = ea_r[...]
        oe_r[...] = jnp.maximum(_mm(jnp.maximum(_mm(eb, w1e_r[...]), 0.0),
                                    w2e_r[...]), 0.0)
        oh_r[...] = jnp.maximum(_mm(jnp.maximum(_mm(eb, w1h_r[...]), 0.0),
                                    w2h_r[...]), 0.0)

    outs = [jax.ShapeDtypeStruct((ne, 64), jnp.float32)] * 2
    return _tc_call(body, (ne // _BE,), [ea, w1e, w2e, w1h, w2h],
                    [_BE, 0, 0, 0, 0], outs, _BE)


def _edge_rel(g, e_prev, w1, b1, w2, b2, mask=None):
    ne = g.shape[0]
    use_mask = mask is not None

    def body(*refs):
        if use_mask:
            (g_r, e_r, mk_r, w1_r, b1_r, w2_r, b2_r, me_r, en_r) = refs
        else:
            (g_r, e_r, w1_r, b1_r, w2_r, b2_r, me_r, en_r) = refs
        eb = e_r[...]
        if use_mask:
            eb = eb * mk_r[...]
        cat = jnp.concatenate([g_r[...], eb], axis=1)
        hidden = jnp.maximum(_mm(cat, w1_r[...]) + b1_r[...][None, :], 0.0)
        m = _mm(hidden, w2_r[...]) + b2_r[...][None, :]
        me_r[...] = m * mk_r[...] if use_mask else m
        en_r[...] = jnp.maximum(m, 0.0)

    arrays = [g, e_prev] + ([mask] if use_mask else []) + [w1, b1, w2, b2]
    flags = [_BE, _BE] + ([_BE] if use_mask else []) + [0, 0, 0, 0]
    outs = [jax.ShapeDtypeStruct((ne, 64), jnp.float32)] * 2
    return _tc_call(body, (ne // _BE,), arrays, flags, outs, _BE)


def _edge_rel_e_only(g, e_prev, w1, b1, w2, b2):
    ne = g.shape[0]

    def body(g_r, e_r, w1_r, b1_r, w2_r, b2_r, en_r):
        cat = jnp.concatenate([g_r[...], e_r[...]], axis=1)
        hidden = jnp.maximum(_mm(cat, w1_r[...]) + b1_r[...][None, :], 0.0)
        m = _mm(hidden, w2_r[...]) + b2_r[...][None, :]
        en_r[...] = jnp.maximum(m, 0.0)

    arrays = [g, e_prev, w1, b1, w2, b2]
    flags = [_BE, _BE, 0, 0, 0, 0]
    outs = [jax.ShapeDtypeStruct((ne, 64), jnp.float32)]
    return _tc_call(body, (ne // _BE,), arrays, flags, outs, _BE)[0]


def _node_obj(x, parts_a, parts_b, wo1, bo1, wo2, bo2):
    def body(x_r, pa_r, pb_r, wo1_r, bo1_r, wo2_r, bo2_r, o_r):
        xb = x_r[...]
        agg = pa_r[0] + pa_r[1] + pb_r[0] + pb_r[1]
        cat = jnp.concatenate([xb, agg], axis=1)
        hidden = jnp.maximum(_mm(cat, wo1_r[...]) + bo1_r[...][None, :], 0.0)
        xn = _mm(hidden, wo2_r[...]) + bo2_r[...][None, :]
        o_r[...] = ALPHA * xb + (1.0 - ALPHA) * jnp.maximum(xn, 0.0)

    def pspec():
        return pl.BlockSpec((2, _BN, 64), lambda i: (0, i, 0))

    in_specs = [_rows(x.shape, _BN), pspec(), pspec(), _full(wo1.shape),
                _full(bo1.shape), _full(wo2.shape), _full(bo2.shape)]
    out_shape = jax.ShapeDtypeStruct((N, H), jnp.float32)
    return pl.pallas_call(
        body, grid=(N // _BN,), in_specs=in_specs,
        out_specs=_rows((N, H), _BN), out_shape=out_shape,
    )(x, parts_a, parts_b, wo1, bo1, wo2, bo2)


def _w_mlp(e0, e1, e2, e3, w1, b1, w2, b2, w3, b3):
    ne = e0.shape[0]

    def body(e0_r, e1_r, e2_r, e3_r, w1_r, b1_r, w2_r, b2_r, w3_r, b3_r,
             ew_r, mk_r):
        cat = jnp.concatenate([e0_r[...], e1_r[...], e2_r[...], e3_r[...]],
                              axis=1)
        h1 = jnp.maximum(_mm(cat, w1_r[...]) + b1_r[...][None, :], 0.0)
        h2 = jnp.maximum(_mm(h1, w2_r[...]) + b2_r[...][None, :], 0.0)
        logit = _mm(h2, w3_r[...]) + b3_r[...][None, :]
        ew = jax.nn.sigmoid(logit)
        ew_r[...] = ew
        mk_r[...] = (ew > 0.5).astype(jnp.float32)

    arrays = [e0, e1, e2, e3, w1, b1, w2, b2, w3, b3]
    flags = [_BE, _BE, _BE, _BE, 0, 0, 0, 0, 0, 0]
    outs = [jax.ShapeDtypeStruct((ne, 1), jnp.float32)] * 2
    return _tc_call(body, (ne // _BE,), arrays, flags, outs, _BE)


def _p_rel(g, e0, e1, e2, e3, mask, w1, b1, w2, b2):
    ne = g.shape[0]

    def body(g_r, e0_r, e1_r, e2_r, e3_r, mk_r, w1_r, b1_r, w2_r, b2_r,
             o_r):
        mk = mk_r[...]
        cat = jnp.concatenate(
            [g_r[...], e0_r[...] * mk, e1_r[...] * mk, e2_r[...] * mk,
             e3_r[...] * mk], axis=1)
        hidden = jnp.maximum(_mm(cat, w1_r[...]) + b1_r[...][None, :], 0.0)
        m = (_mm(hidden, w2_r[...]) + b2_r[...][None, :]) * mk
        o_r[...] = jnp.concatenate(
            [m, jnp.zeros((m.shape[0], 15), jnp.float32)], axis=1)

    arrays = [g, e0, e1, e2, e3, mask, w1, b1, w2, b2]
    flags = [_BE, _BE, _BE, _BE, _BE, _BE, 0, 0, 0, 0]
    outs = [jax.ShapeDtypeStruct((ne, 16), jnp.float32)]
    return _tc_call(body, (ne // _BE,), arrays, flags, outs, _BE)[0]


def _final_node(h, parts_a, parts_b, bw1, bb1, bw2, bb2, bw3, bb3,
                xw1, xb1, xw2, xb2, xw3, xb3,
                pwh, pwa, pb1, pw2, pb2):
    def body(h_r, pa_r, pb_r, bw1_r, bb1_r, bw2_r, bb2_r, bw3_r, bb3_r,
             xw1_r, xb1_r, xw2_r, xb2_r, xw3_r, xb3_r,
             pwh_r, pwa_r, pb1_r, pw2_r, pb2_r,
             beta_r, hout_r, trk_r):
        hb = h_r[...]
        agg = (pa_r[0] + pa_r[1] + pb_r[0] + pb_r[1])[:, 0:1]
        b1h = jnp.maximum(_mm(hb, bw1_r[...]) + bb1_r[...][None, :], 0.0)
        b2h = jnp.maximum(_mm(b1h, bw2_r[...]) + bb2_r[...][None, :], 0.0)
        beta_r[...] = jax.nn.sigmoid(_mm(b2h, bw3_r[...]) +
                                     bb3_r[...][None, :]) + 1e-08
        x1h = jnp.maximum(_mm(hb, xw1_r[...]) + xb1_r[...][None, :], 0.0)
        x2h = jnp.maximum(_mm(x1h, xw2_r[...]) + xb2_r[...][None, :], 0.0)
        hout_r[...] = _mm(x2h, xw3_r[...]) + xb3_r[...][None, :]
        p1h = jnp.maximum(_mm(hb, pwh_r[...]) + agg * pwa_r[...][None, :] +
                          pb1_r[...][None, :], 0.0)
        trk_r[...] = _mm(p1h, pw2_r[...]) + pb2_r[...][None, :]

    def pspec():
        return pl.BlockSpec((2, _BN, 16), lambda i: (0, i, 0))

    weights = [bw1, bb1, bw2, bb2, bw3, bb3, xw1, xb1, xw2, xb2, xw3, xb3,
               pwh, pwa, pb1, pw2, pb2]
    in_specs = [_rows(h.shape, _BN), pspec(), pspec()] + \
        [_full(w.shape) for w in weights]
    outs = [jax.ShapeDtypeStruct((N, 1), jnp.float32),
            jax.ShapeDtypeStruct((N, 24), jnp.float32),
            jax.ShapeDtypeStruct((N, 1), jnp.float32)]
    out_specs = [_rows(o.shape, _BN) for o in outs]
    return pl.pallas_call(
        body, grid=(N // _BN,), in_specs=in_specs, out_specs=out_specs,
        out_shape=outs,
    )(h, parts_a, parts_b, *weights)


def _resin(h, e_halves, idx3ds, dst3ds, zeros64, layer_params, masks=None,
           skip_last_node=False):
    e_lists = [[eh] for eh in e_halves]
    last = len(layer_params) - 1
    for li, p in enumerate(layer_params):
        wr1, br1 = p["rel"][0]
        wr2, br2 = p["rel"][1]
        gs = [_sc_gather(h, idx3ds[k], 2 * EHS[k]).reshape(EHS[k], 2 * H)
              for k in range(2)]
        if skip_last_node and li == last:
            for k in range(2):
                e_lists[k].append(
                    _edge_rel_e_only(gs[k], e_lists[k][-1], wr1, br1,
                                     wr2, br2))
            break
        wo1, bo1 = p["obj"][0]
        wo2, bo2 = p["obj"][1]
        parts = []
        for k in range(2):
            m_eff, e_next = _edge_rel(
                gs[k], e_lists[k][-1], wr1, br1, wr2, br2,
                mask=None if masks is None else masks[k])
            parts.append(_sc_scatter(m_eff, dst3ds[k], zeros64, 64))
            e_lists[k].append(e_next)
        h = _node_obj(h, parts[0], parts[1], wo1, bo1, wo2, bo2)
    return h, e_lists


def kernel(x, edge_index, edge_attr, params):
    offs = [0, EHS[0]]
    idx3ds = [edge_index[:, o:o + eh].T.reshape(_NW, -1, 125)
              for o, eh in zip(offs, EHS)]
    dst3ds = [edge_index[1, o:o + eh].reshape(_NW, -1, 125)
              for o, eh in zip(offs, EHS)]
    zeros64 = jnp.zeros((N, 64), jnp.float32)
    zeros16 = jnp.zeros((N, 16), jnp.float32)

    h_ec, h_hc = _node_enc(x, params["ec_node_enc"][0][0],
                           params["ec_node_enc"][1][0],
                           params["hc_node_enc"][0][0],
                           params["hc_node_enc"][1][0])
    e_enc = [_edge_enc(edge_attr[o:o + eh], params["ec_edge_enc"][0][0],
                       params["ec_edge_enc"][1][0],
                       params["hc_edge_enc"][0][0],
                       params["hc_edge_enc"][1][0])
             for o, eh in zip(offs, EHS)]

    _, e_lists_ec = _resin(h_ec, [e_enc[0][0], e_enc[1][0]], idx3ds, dst3ds,
                           zeros64, params["ec_resin"], skip_last_node=True)
    (ww1, wb1), (ww2, wb2), (ww3, wb3) = params["W_mlp"]
    ew_mk = [_w_mlp(*e_lists_ec[k], ww1, wb1, ww2, wb2, ww3, wb3)
             for k in range(2)]
    masks = [ew_mk[0][1], ew_mk[1][1]]

    h_hc, e_lists_hc = _resin(h_hc, [e_enc[0][1], e_enc[1][1]], idx3ds,
                              dst3ds, zeros64, params["hc_resin"],
                              masks=masks)

    pr1, prb1 = params["P_in"]["rel"][0]
    pr2, prb2 = params["P_in"]["rel"][1]
    p_parts = []
    for k in range(2):
        g_p = _sc_gather(h_hc, idx3ds[k], 2 * EHS[k]).reshape(EHS[k], 2 * H)
        m_p = _p_rel(g_p, *e_lists_hc[k], masks[k], pr1, prb1, pr2, prb2)
        p_parts.append(_sc_scatter(m_p, dst3ds[k], zeros16, 16))

    (bw1, bb1), (bw2, bb2), (bw3, bb3) = params["B_mlp"]
    (xw1, xb1), (xw2, xb2), (xw3, xb3) = params["X_mlp"]
    po1, pob1 = params["P_in"]["obj"][0]
    po2, pob2 = params["P_in"]["obj"][1]
    beta, h_out, track = _final_node(
        h_hc, p_parts[0], p_parts[1], bw1, bb1, bw2, bb2, bw3, bb3,
        xw1, xb1, xw2, xb2, xw3, xb3,
        po1[0:H], po1[H], pob1, po2, pob2)

    edge_weights = jnp.concatenate([ew_mk[0][0], ew_mk[1][0]], axis=0)
    return edge_weights, h_out, beta, track


# speedup vs baseline: 1.7744x; 1.0086x over previous
import functools

import jax
import jax.numpy as jnp
from jax import lax
from jax.experimental import pallas as pl
from jax.experimental.pallas import tpu as pltpu
from jax.experimental.pallas import tpu_sc as plsc

N = 10000
E = 160000
H = 128
ALPHA = 0.5
EHS = (96000, 64000)

_NC = 2
_NS = 16
_NW = _NC * _NS


def _mm(a, b):
    return jax.lax.dot_general(a, b, (((1,), (0,)), ((), ())),
                               preferred_element_type=jnp.float32)


@functools.partial(jax.jit, static_argnames=("ni",))
def _sc_gather(table, idx3d, ni):
    per_w = ni // _NW
    chunk = 1000
    n_chunks = per_w // chunk
    mesh = plsc.VectorSubcoreMesh(core_axis_name="c", subcore_axis_name="s")

    @functools.partial(
        pl.kernel, mesh=mesh,
        compiler_params=pltpu.CompilerParams(use_tc_tiling_on_sc=False),
        out_type=jax.ShapeDtypeStruct((ni, H), jnp.float32),
        scratch_types=[
            pltpu.VMEM((8, 125), jnp.int32),
            pltpu.VMEM((chunk, H), jnp.float32),
            pltpu.SemaphoreType.DMA,
        ],
    )
    def k(table_hbm, idx_hbm, out_hbm, idx_v, rows_v, sem):
        wid = lax.axis_index("s") * _NC + lax.axis_index("c")
        my_idx = idx_hbm.at[wid]

        def body(c, _):
            pltpu.sync_copy(my_idx.at[pl.ds(c * 8, 8)], idx_v)
            cps = [
                pltpu.async_copy(table_hbm.at[idx_v.at[j]],
                                 rows_v.at[pl.ds(j * 125, 125)], sem)
                for j in range(8)
            ]
            for cp in cps:
                cp.wait()
            pltpu.sync_copy(rows_v,
                            out_hbm.at[pl.ds(wid * per_w + c * chunk, chunk)])
            return 0

        lax.fori_loop(0, n_chunks, body, 0)

    return k(table, idx3d)


@functools.partial(jax.jit, static_argnames=("d",))
def _sc_scatter(msgs, dst3d, init, d):
    ne = msgs.shape[0]
    chained = init.ndim == 3
    per_w = ne // _NW
    chunk = 1000
    n_chunks = per_w // chunk
    mesh = plsc.VectorSubcoreMesh(core_axis_name="c", subcore_axis_name="s")

    @functools.partial(
        pl.kernel, mesh=mesh,
        compiler_params=pltpu.CompilerParams(use_tc_tiling_on_sc=False),
        out_type=jax.ShapeDtypeStruct((_NC, N, d), jnp.float32),
        scratch_types=[
            pltpu.VMEM((8, 125), jnp.int32),
            pltpu.VMEM((chunk, d), jnp.float32),
            pltpu.SemaphoreType.DMA,
            pltpu.VMEM_SHARED((N, d), jnp.float32),
        ],
    )
    def k(msg_hbm, dst_hbm, init_hbm, out_hbm, idx_v, m_v, sem, acc):
        cid = lax.axis_index("c")
        sid = lax.axis_index("s")
        wid = sid * _NC + cid
        base = wid * per_w
        my_dst = dst_hbm.at[wid]
        src = init_hbm.at[cid] if chained else init_hbm

        @pl.when(sid < 15)
        def _():
            s = pl.ds(sid * 640, 640)
            pltpu.sync_copy(src.at[s], acc.at[s])

        @pl.when(sid == 15)
        def _():
            s = pl.ds(9600, 400)
            pltpu.sync_copy(src.at[s], acc.at[s])

        plsc.subcore_barrier()

        def body(c, _):
            pltpu.sync_copy(my_dst.at[pl.ds(c * 8, 8)], idx_v)
            pltpu.sync_copy(msg_hbm.at[pl.ds(base + c * chunk, chunk)], m_v)
            for j in range(8):
                pltpu.sync_copy(m_v.at[pl.ds(j * 125, 125)],
                                acc.at[idx_v.at[j]], add=True)
            return 0

        lax.fori_loop(0, n_chunks, body, 0)

        plsc.subcore_barrier()
        @pl.when(sid < 15)
        def _():
            s = pl.ds(sid * 640, 640)
            pltpu.sync_copy(acc.at[s], out_hbm.at[cid].at[s])

        @pl.when(sid == 15)
        def _():
            s = pl.ds(9600, 400)
            pltpu.sync_copy(acc.at[s], out_hbm.at[cid].at[s])

    return k(msgs, dst3d, init)


_BN = 1000
_BE = 2000


def _full(shape):
    nd = len(shape)
    return pl.BlockSpec(shape, lambda i, _nd=nd: (0,) * _nd)


def _rows(shape, bs):
    blk = (bs,) + shape[1:]
    nd = len(shape)
    return pl.BlockSpec(blk, lambda i, _nd=nd: (i,) + (0,) * (_nd - 1))


def _tc_call(body, grid, in_arrays, in_row_flags, out_shapes, out_bs):
    in_specs = [
        _rows(a.shape, bs) if bs else _full(a.shape)
        for a, bs in zip(in_arrays, in_row_flags)
    ]
    out_specs = [_rows(s.shape, out_bs) for s in out_shapes]
    return pl.pallas_call(
        body, grid=grid, in_specs=in_specs, out_specs=out_specs,
        out_shape=out_shapes,
    )(*in_arrays)


def _node_enc(x, w1e, w2e, w1h, w2h):
    def body(x_r, w1e_r, w2e_r, w1h_r, w2h_r, oe_r, oh_r):
        xb = x_r[...]
        oe_r[...] = jnp.maximum(_mm(jnp.maximum(_mm(xb, w1e_r[...]), 0.0),
                                    w2e_r[...]), 0.0)
        oh_r[...] = jnp.maximum(_mm(jnp.maximum(_mm(xb, w1h_r[...]), 0.0),
                                    w2h_r[...]), 0.0)

    outs = [jax.ShapeDtypeStruct((N, H), jnp.float32)] * 2
    return _tc_call(body, (N // _BN,), [x, w1e, w2e, w1h, w2h],
                    [_BN, 0, 0, 0, 0], outs, _BN)


def _edge_enc(ea, w1e, w2e, w1h, w2h):
    ne = ea.shape[0]

    def body(ea_r, w1e_r, w2e_r, w1h_r, w2h_r, oe_r, oh_r):
        eb = ea_r[...]
        oe_r[...] = jnp.maximum(_mm(jnp.maximum(_mm(eb, w1e_r[...]), 0.0),
                                    w2e_r[...]), 0.0)
        oh_r[...] = jnp.maximum(_mm(jnp.maximum(_mm(eb, w1h_r[...]), 0.0),
                                    w2h_r[...]), 0.0)

    outs = [jax.ShapeDtypeStruct((ne, 64), jnp.float32)] * 2
    return _tc_call(body, (ne // _BE,), [ea, w1e, w2e, w1h, w2h],
                    [_BE, 0, 0, 0, 0], outs, _BE)


def _edge_rel(g, e_prev, w1, b1, w2, b2, mask=None):
    ne = g.shape[0]
    use_mask = mask is not None

    def body(*refs):
        if use_mask:
            (g_r, e_r, mk_r, w1_r, b1_r, w2_r, b2_r, me_r, en_r) = refs
        else:
            (g_r, e_r, w1_r, b1_r, w2_r, b2_r, me_r, en_r) = refs
        eb = e_r[...]
        if use_mask:
            eb = eb * mk_r[...]
        cat = jnp.concatenate([g_r[...], eb], axis=1)
        hidden = jnp.maximum(_mm(cat, w1_r[...]) + b1_r[...][None, :], 0.0)
        m = _mm(hidden, w2_r[...]) + b2_r[...][None, :]
        me_r[...] = m * mk_r[...] if use_mask else m
        en_r[...] = jnp.maximum(m, 0.0)

    arrays = [g, e_prev] + ([mask] if use_mask else []) + [w1, b1, w2, b2]
    flags = [_BE, _BE] + ([_BE] if use_mask else []) + [0, 0, 0, 0]
    outs = [jax.ShapeDtypeStruct((ne, 64), jnp.float32)] * 2
    return _tc_call(body, (ne // _BE,), arrays, flags, outs, _BE)


def _edge_rel_e_only(g, e_prev, w1, b1, w2, b2):
    ne = g.shape[0]

    def body(g_r, e_r, w1_r, b1_r, w2_r, b2_r, en_r):
        cat = jnp.concatenate([g_r[...], e_r[...]], axis=1)
        hidden = jnp.maximum(_mm(cat, w1_r[...]) + b1_r[...][None, :], 0.0)
        m = _mm(hidden, w2_r[...]) + b2_r[...][None, :]
        en_r[...] = jnp.maximum(m, 0.0)

    arrays = [g, e_prev, w1, b1, w2, b2]
    flags = [_BE, _BE, 0, 0, 0, 0]
    outs = [jax.ShapeDtypeStruct((ne, 64), jnp.float32)]
    return _tc_call(body, (ne // _BE,), arrays, flags, outs, _BE)[0]


def _node_obj(x, parts, wo1, bo1, wo2, bo2):
    def body(x_r, p_r, wo1_r, bo1_r, wo2_r, bo2_r, o_r):
        xb = x_r[...]
        agg = p_r[0] + p_r[1]
        cat = jnp.concatenate([xb, agg], axis=1)
        hidden = jnp.maximum(_mm(cat, wo1_r[...]) + bo1_r[...][None, :], 0.0)
        xn = _mm(hidden, wo2_r[...]) + bo2_r[...][None, :]
        o_r[...] = ALPHA * xb + (1.0 - ALPHA) * jnp.maximum(xn, 0.0)

    def pspec():
        return pl.BlockSpec((2, _BN, 64), lambda i: (0, i, 0))

    in_specs = [_rows(x.shape, _BN), pspec(), _full(wo1.shape),
                _full(bo1.shape), _full(wo2.shape), _full(bo2.shape)]
    out_shape = jax.ShapeDtypeStruct((N, H), jnp.float32)
    return pl.pallas_call(
        body, grid=(N // _BN,), in_specs=in_specs,
        out_specs=_rows((N, H), _BN), out_shape=out_shape,
    )(x, parts, wo1, bo1, wo2, bo2)


def _w_mlp(e0, e1, e2, e3, w1, b1, w2, b2, w3, b3):
    ne = e0.shape[0]

    def body(e0_r, e1_r, e2_r, e3_r, w1_r, b1_r, w2_r, b2_r, w3_r, b3_r,
             ew_r, mk_r):
        cat = jnp.concatenate([e0_r[...], e1_r[...], e2_r[...], e3_r[...]],
                              axis=1)
        h1 = jnp.maximum(_mm(cat, w1_r[...]) + b1_r[...][None, :], 0.0)
        h2 = jnp.maximum(_mm(h1, w2_r[...]) + b2_r[...][None, :], 0.0)
        logit = _mm(h2, w3_r[...]) + b3_r[...][None, :]
        ew = jax.nn.sigmoid(logit)
        ew_r[...] = ew
        mk_r[...] = (ew > 0.5).astype(jnp.float32)

    arrays = [e0, e1, e2, e3, w1, b1, w2, b2, w3, b3]
    flags = [_BE, _BE, _BE, _BE, 0, 0, 0, 0, 0, 0]
    outs = [jax.ShapeDtypeStruct((ne, 1), jnp.float32)] * 2
    return _tc_call(body, (ne // _BE,), arrays, flags, outs, _BE)


def _p_rel(g, e0, e1, e2, e3, mask, w1, b1, w2, b2):
    ne = g.shape[0]

    def body(g_r, e0_r, e1_r, e2_r, e3_r, mk_r, w1_r, b1_r, w2_r, b2_r,
             o_r):
        mk = mk_r[...]
        cat = jnp.concatenate(
            [g_r[...], e0_r[...] * mk, e1_r[...] * mk, e2_r[...] * mk,
             e3_r[...] * mk], axis=1)
        hidden = jnp.maximum(_mm(cat, w1_r[...]) + b1_r[...][None, :], 0.0)
        m = (_mm(hidden, w2_r[...]) + b2_r[...][None, :]) * mk
        o_r[...] = jnp.concatenate(
            [m, jnp.zeros((m.shape[0], 15), jnp.float32)], axis=1)

    arrays = [g, e0, e1, e2, e3, mask, w1, b1, w2, b2]
    flags = [_BE, _BE, _BE, _BE, _BE, _BE, 0, 0, 0, 0]
    outs = [jax.ShapeDtypeStruct((ne, 16), jnp.float32)]
    return _tc_call(body, (ne // _BE,), arrays, flags, outs, _BE)[0]


def _final_node(h, parts, bw1, bb1, bw2, bb2, bw3, bb3,
                xw1, xb1, xw2, xb2, xw3, xb3,
                pwh, pwa, pb1, pw2, pb2):
    def body(h_r, p_r, bw1_r, bb1_r, bw2_r, bb2_r, bw3_r, bb3_r,
             xw1_r, xb1_r, xw2_r, xb2_r, xw3_r, xb3_r,
             pwh_r, pwa_r, pb1_r, pw2_r, pb2_r,
             beta_r, hout_r, trk_r):
        hb = h_r[...]
        agg = (p_r[0] + p_r[1])[:, 0:1]
        b1h = jnp.maximum(_mm(hb, bw1_r[...]) + bb1_r[...][None, :], 0.0)
        b2h = jnp.maximum(_mm(b1h, bw2_r[...]) + bb2_r[...][None, :], 0.0)
        beta_r[...] = jax.nn.sigmoid(_mm(b2h, bw3_r[...]) +
                                     bb3_r[...][None, :]) + 1e-08
        x1h = jnp.maximum(_mm(hb, xw1_r[...]) + xb1_r[...][None, :], 0.0)
        x2h = jnp.maximum(_mm(x1h, xw2_r[...]) + xb2_r[...][None, :], 0.0)
        hout_r[...] = _mm(x2h, xw3_r[...]) + xb3_r[...][None, :]
        p1h = jnp.maximum(_mm(hb, pwh_r[...]) + agg * pwa_r[...][None, :] +
                          pb1_r[...][None, :], 0.0)
        trk_r[...] = _mm(p1h, pw2_r[...]) + pb2_r[...][None, :]

    def pspec():
        return pl.BlockSpec((2, _BN, 16), lambda i: (0, i, 0))

    weights = [bw1, bb1, bw2, bb2, bw3, bb3, xw1, xb1, xw2, xb2, xw3, xb3,
               pwh, pwa, pb1, pw2, pb2]
    in_specs = [_rows(h.shape, _BN), pspec()] + \
        [_full(w.shape) for w in weights]
    outs = [jax.ShapeDtypeStruct((N, 1), jnp.float32),
            jax.ShapeDtypeStruct((N, 24), jnp.float32),
            jax.ShapeDtypeStruct((N, 1), jnp.float32)]
    out_specs = [_rows(o.shape, _BN) for o in outs]
    return pl.pallas_call(
        body, grid=(N // _BN,), in_specs=in_specs, out_specs=out_specs,
        out_shape=outs,
    )(h, parts, *weights)


def _resin(h, e_halves, idx3ds, dst3ds, zeros64, layer_params, masks=None,
           skip_last_node=False):
    e_lists = [[eh] for eh in e_halves]
    last = len(layer_params) - 1
    for li, p in enumerate(layer_params):
        wr1, br1 = p["rel"][0]
        wr2, br2 = p["rel"][1]
        gs = [_sc_gather(h, idx3ds[k], 2 * EHS[k]).reshape(EHS[k], 2 * H)
              for k in range(2)]
        if skip_last_node and li == last:
            for k in range(2):
                e_lists[k].append(
                    _edge_rel_e_only(gs[k], e_lists[k][-1], wr1, br1,
                                     wr2, br2))
            break
        wo1, bo1 = p["obj"][0]
        wo2, bo2 = p["obj"][1]
        parts = zeros64
        for k in range(2):
            m_eff, e_next = _edge_rel(
                gs[k], e_lists[k][-1], wr1, br1, wr2, br2,
                mask=None if masks is None else masks[k])
            parts = _sc_scatter(m_eff, dst3ds[k], parts, 64)
            e_lists[k].append(e_next)
        h = _node_obj(h, parts, wo1, bo1, wo2, bo2)
    return h, e_lists


def kernel(x, edge_index, edge_attr, params):
    offs = [0, EHS[0]]
    idx3ds = [edge_index[:, o:o + eh].T.reshape(_NW, -1, 125)
              for o, eh in zip(offs, EHS)]
    dst3ds = [edge_index[1, o:o + eh].reshape(_NW, -1, 125)
              for o, eh in zip(offs, EHS)]
    zeros64 = jnp.zeros((N, 64), jnp.float32)
    zeros16 = jnp.zeros((N, 16), jnp.float32)

    h_ec, h_hc = _node_enc(x, params["ec_node_enc"][0][0],
                           params["ec_node_enc"][1][0],
                           params["hc_node_enc"][0][0],
                           params["hc_node_enc"][1][0])
    e_enc = [_edge_enc(edge_attr[o:o + eh], params["ec_edge_enc"][0][0],
                       params["ec_edge_enc"][1][0],
                       params["hc_edge_enc"][0][0],
                       params["hc_edge_enc"][1][0])
             for o, eh in zip(offs, EHS)]

    _, e_lists_ec = _resin(h_ec, [e_enc[0][0], e_enc[1][0]], idx3ds, dst3ds,
                           zeros64, params["ec_resin"], skip_last_node=True)
    (ww1, wb1), (ww2, wb2), (ww3, wb3) = params["W_mlp"]
    ew_mk = [_w_mlp(*e_lists_ec[k], ww1, wb1, ww2, wb2, ww3, wb3)
             for k in range(2)]
    masks = [ew_mk[0][1], ew_mk[1][1]]

    h_hc, e_lists_hc = _resin(h_hc, [e_enc[0][1], e_enc[1][1]], idx3ds,
                              dst3ds, zeros64, params["hc_resin"],
                              masks=masks)

    pr1, prb1 = params["P_in"]["rel"][0]
    pr2, prb2 = params["P_in"]["rel"][1]
    p_part = zeros16
    for k in range(2):
        g_p = _sc_gather(h_hc, idx3ds[k], 2 * EHS[k]).reshape(EHS[k], 2 * H)
        m_p = _p_rel(g_p, *e_lists_hc[k], masks[k], pr1, prb1, pr2, prb2)
        p_part = _sc_scatter(m_p, dst3ds[k], p_part, 16)

    (bw1, bb1), (bw2, bb2), (bw3, bb3) = params["B_mlp"]
    (xw1, xb1), (xw2, xb2), (xw3, xb3) = params["X_mlp"]
    po1, pob1 = params["P_in"]["obj"][0]
    po2, pob2 = params["P_in"]["obj"][1]
    beta, h_out, track = _final_node(
        h_hc, p_part, bw1, bb1, bw2, bb2, bw3, bb3,
        xw1, xb1, xw2, xb2, xw3, xb3,
        po1[0:H], po1[H], pob1, po2, pob2)

    edge_weights = jnp.concatenate([ew_mk[0][0], ew_mk[1][0]], axis=0)
    return edge_weights, h_out, beta, track
